# baseline (speedup 1.0000x reference)
import jax
import jax.numpy as jnp
from jax import lax
from jax.experimental import pallas as pl
from jax.experimental.pallas import tpu as pltpu
from jax.experimental.pallas import tpu_sc as plsc

V = 1000000
D = 64
B = 16384
R = 8192
NSTEP = (V + R - 1) // R
NSLAB = NSTEP * (R // 128)
NC = 2
NS = 16
NW = NC * NS
BPW = B // NW
CHUNK = 128
NCHUNK = BPW // CHUNK


def _tc_body(ia_ref, w_ref, arow_ref, o_ref):
    w = w_ref[...]
    a = arow_ref[ia_ref[1], :]
    p = jnp.dot(w, a, preferred_element_type=jnp.float32)
    q = jnp.sum(w * w, axis=1)
    o_ref[:, 0, :] = p.reshape(R // 128, 128)
    o_ref[:, 1, :] = q.reshape(R // 128, 128)


def _nrsqrt(s):
    i = plsc.bitcast(s, jnp.int32)
    i = jnp.int32(0x5F3759DF) - lax.shift_right_logical(i, jnp.int32(1))
    y = plsc.bitcast(i, jnp.float32)
    for _ in range(3):
        y = y * (jnp.float32(1.5) - jnp.float32(0.5) * s * y * y)
    return y


def _sc_body(pq_hbm, idx_hbm, iat_hbm, rva_hbm, out_hbm,
             idx_v, sidx_v, iat_v, rva_v, at_v, buf, out_v, sem, sem_a):
    wid = lax.axis_index("s") * NC + lax.axis_index("c")

    pltpu.sync_copy(idx_hbm.at[pl.ds(wid * 4, 4)], idx_v)
    pltpu.sync_copy(iat_hbm, iat_v)
    pltpu.sync_copy(rva_hbm, rva_v)

    h_anchor = pltpu.async_copy(pq_hbm.at[iat_v], at_v, sem_a)

    for j in range(4):
        for k in range(8):
            v = idx_v[j, pl.ds(k * 16, 16)]
            sidx_v[j, pl.ds(k * 16, 16)] = lax.shift_right_logical(
                v, jnp.int32(7))

    zero = jnp.zeros((16,), jnp.int32)
    one = jnp.full((16,), 1, jnp.int32)
    c127 = jnp.full((16,), 127, jnp.int32)
    lanes = lax.iota(jnp.int32, 16)

    h_anchor.wait()
    rva = rva_v[pl.ds(0, 16)]
    sa_v = jnp.maximum(plsc.load_gather(at_v, [zero, one, rva]),
                       jnp.float32(1e-16))

    for c in range(NCHUNK):
        pltpu.async_copy(pq_hbm.at[sidx_v.at[c, pl.ds(0, CHUNK)]],
                         buf, sem).wait()
        for g in range(CHUNK // 16):
            pos = g * 16 + lanes
            lv = jnp.bitwise_and(idx_v[c, pl.ds(g * 16, 16)], c127)
            p_vals = plsc.load_gather(buf, [pos, zero, lv])
            q_vals = plsc.load_gather(buf, [pos, one, lv])
            r = _nrsqrt(sa_v * jnp.maximum(q_vals, jnp.float32(1e-16)))
            out_v[pl.ds(c * CHUNK + g * 16, 16)] = p_vals * r

    pltpu.sync_copy(out_v, out_hbm.at[pl.ds(wid * BPW, BPW)])


def kernel(x, weight):
    ia = x[0, 0].astype(jnp.int32)
    ia_arr = jnp.stack([ia >> 3, ia & 7])
    idx = x[:, 1].astype(jnp.int32).reshape(NW * 4, 128)
    iat = jnp.broadcast_to((ia >> 7)[None], (8,))
    rva = jnp.broadcast_to((ia & 127)[None], (16,))

    pq = pl.pallas_call(
        _tc_body,
        grid_spec=pltpu.PrefetchScalarGridSpec(
            num_scalar_prefetch=1,
            grid=(NSTEP,),
            in_specs=[
                pl.BlockSpec((R, D), lambda i, ia_r: (i, 0)),
                pl.BlockSpec((8, D), lambda i, ia_r: (ia_r[0], 0)),
            ],
            out_specs=pl.BlockSpec((R // 128, 2, 128),
                                   lambda i, ia_r: (i, 0, 0)),
        ),
        out_shape=jax.ShapeDtypeStruct((NSLAB, 2, 128), jnp.float32),
    )(ia_arr, weight, weight)

    run = pl.kernel(
        _sc_body,
        out_type=jax.ShapeDtypeStruct((B,), jnp.float32),
        mesh=plsc.VectorSubcoreMesh(core_axis_name="c", subcore_axis_name="s",
                                    num_cores=NC, num_subcores=NS),
        compiler_params=pltpu.CompilerParams(needs_layout_passes=False),
        scratch_types=[
            pltpu.VMEM((4, 128), jnp.int32),
            pltpu.VMEM((4, 128), jnp.int32),
            pltpu.VMEM((8,), jnp.int32),
            pltpu.VMEM((16,), jnp.int32),
            pltpu.VMEM((8, 2, 128), jnp.float32),
            pltpu.VMEM((CHUNK, 2, 128), jnp.float32),
            pltpu.VMEM((BPW,), jnp.float32),
            pltpu.SemaphoreType.DMA,
            pltpu.SemaphoreType.DMA,
        ],
    )
    return run(pq, idx, iat, rva)


# speedup vs baseline: 1.0072x; 1.0072x over previous
import jax
import jax.numpy as jnp
from jax import lax
from jax.experimental import pallas as pl
from jax.experimental.pallas import tpu as pltpu
from jax.experimental.pallas import tpu_sc as plsc

V = 1000000
D = 64
B = 16384
R = 32768
NSTEP = (V + R - 1) // R
NSLAB = NSTEP * (R // 128)
NC = 2
NS = 16
NW = NC * NS
BPW = B // NW
CHUNK = 128
NCHUNK = BPW // CHUNK


def _tc_body(ia_ref, w_ref, arow_ref, o_ref):
    w = w_ref[...]
    a = arow_ref[ia_ref[1], :]
    p = jnp.dot(w, a, preferred_element_type=jnp.float32)
    q = jnp.sum(w * w, axis=1)
    o_ref[:, 0, :] = p.reshape(R // 128, 128)
    o_ref[:, 1, :] = q.reshape(R // 128, 128)


def _nrsqrt(s):
    i = plsc.bitcast(s, jnp.int32)
    i = jnp.int32(0x5F3759DF) - lax.shift_right_logical(i, jnp.int32(1))
    y = plsc.bitcast(i, jnp.float32)
    for _ in range(3):
        y = y * (jnp.float32(1.5) - jnp.float32(0.5) * s * y * y)
    return y


def _sc_body(pq_hbm, idx_hbm, iat_hbm, rva_hbm, out_hbm,
             idx_v, sidx_v, iat_v, rva_v, at_v, buf, out_v, sem, sem_a):
    wid = lax.axis_index("s") * NC + lax.axis_index("c")

    pltpu.sync_copy(idx_hbm.at[pl.ds(wid * 4, 4)], idx_v)
    pltpu.sync_copy(iat_hbm, iat_v)
    pltpu.sync_copy(rva_hbm, rva_v)

    h_anchor = pltpu.async_copy(pq_hbm.at[iat_v], at_v, sem_a)

    for j in range(4):
        for k in range(8):
            v = idx_v[j, pl.ds(k * 16, 16)]
            sidx_v[j, pl.ds(k * 16, 16)] = lax.shift_right_logical(
                v, jnp.int32(7))

    zero = jnp.zeros((16,), jnp.int32)
    one = jnp.full((16,), 1, jnp.int32)
    c127 = jnp.full((16,), 127, jnp.int32)
    lanes = lax.iota(jnp.int32, 16)

    h_anchor.wait()
    rva = rva_v[pl.ds(0, 16)]
    sa_v = jnp.maximum(plsc.load_gather(at_v, [zero, one, rva]),
                       jnp.float32(1e-16))

    for c in range(NCHUNK):
        pltpu.async_copy(pq_hbm.at[sidx_v.at[c, pl.ds(0, CHUNK)]],
                         buf, sem).wait()
        for g in range(CHUNK // 16):
            pos = g * 16 + lanes
            lv = jnp.bitwise_and(idx_v[c, pl.ds(g * 16, 16)], c127)
            p_vals = plsc.load_gather(buf, [pos, zero, lv])
            q_vals = plsc.load_gather(buf, [pos, one, lv])
            r = _nrsqrt(sa_v * jnp.maximum(q_vals, jnp.float32(1e-16)))
            out_v[pl.ds(c * CHUNK + g * 16, 16)] = p_vals * r

    pltpu.sync_copy(out_v, out_hbm.at[pl.ds(wid * BPW, BPW)])


def kernel(x, weight):
    ia = x[0, 0].astype(jnp.int32)
    ia_arr = jnp.stack([ia >> 3, ia & 7])
    idx = x[:, 1].astype(jnp.int32).reshape(NW * 4, 128)
    iat = jnp.broadcast_to((ia >> 7)[None], (8,))
    rva = jnp.broadcast_to((ia & 127)[None], (16,))

    pq = pl.pallas_call(
        _tc_body,
        grid_spec=pltpu.PrefetchScalarGridSpec(
            num_scalar_prefetch=1,
            grid=(NSTEP,),
            in_specs=[
                pl.BlockSpec((R, D), lambda i, ia_r: (i, 0)),
                pl.BlockSpec((8, D), lambda i, ia_r: (ia_r[0], 0)),
            ],
            out_specs=pl.BlockSpec((R // 128, 2, 128),
                                   lambda i, ia_r: (i, 0, 0)),
        ),
        out_shape=jax.ShapeDtypeStruct((NSLAB, 2, 128), jnp.float32),
    )(ia_arr, weight, weight)

    run = pl.kernel(
        _sc_body,
        out_type=jax.ShapeDtypeStruct((B,), jnp.float32),
        mesh=plsc.VectorSubcoreMesh(core_axis_name="c", subcore_axis_name="s",
                                    num_cores=NC, num_subcores=NS),
        compiler_params=pltpu.CompilerParams(needs_layout_passes=False),
        scratch_types=[
            pltpu.VMEM((4, 128), jnp.int32),
            pltpu.VMEM((4, 128), jnp.int32),
            pltpu.VMEM((8,), jnp.int32),
            pltpu.VMEM((16,), jnp.int32),
            pltpu.VMEM((8, 2, 128), jnp.float32),
            pltpu.VMEM((CHUNK, 2, 128), jnp.float32),
            pltpu.VMEM((BPW,), jnp.float32),
            pltpu.SemaphoreType.DMA,
            pltpu.SemaphoreType.DMA,
        ],
    )
    return run(pq, idx, iat, rva)
